# baseline (speedup 1.0000x reference)
import functools

import jax
import jax.numpy as jnp
from jax import lax
from jax.experimental import pallas as pl
from jax.experimental.pallas import tpu as pltpu
from jax.experimental.pallas import tpu_sc as plsc


def kernel(x, symbol_library):
    batch, seq_len, d_model = x.shape
    num_workers = 32
    rows_per_worker = seq_len // num_workers
    chunk = 16
    n_chunks = rows_per_worker // chunk
    nbuf = 2

    half = seq_len // 2
    sp_workers = 4
    sp_rows = 352
    sp_covered = sp_workers * sp_rows
    tail_sid = sp_covered // rows_per_worker

    mesh = plsc.VectorSubcoreMesh(core_axis_name="c", subcore_axis_name="s")

    @functools.partial(
        pl.kernel,
        mesh=mesh,
        out_type=jax.ShapeDtypeStruct((batch, seq_len, d_model), x.dtype),
        scratch_types=[
            pltpu.VMEM((nbuf, chunk, d_model), jnp.float32),
            pltpu.VMEM_SHARED((sp_covered, d_model), jnp.float32),
            pltpu.SemaphoreType.DMA,
            pltpu.SemaphoreType.DMA,
            pltpu.SemaphoreType.DMA,
            pltpu.SemaphoreType.DMA,
            pltpu.SemaphoreType.DMA,
        ],
    )
    def body(table_hbm, out_hbm, bufs, spmem, rsem, wsem, srsem, swsem, lsem):
        cid = lax.axis_index("c")
        sid = lax.axis_index("s")
        wid = cid * 16 + sid
        base = wid * rows_per_worker

        sp_tab = cid * half + sid * sp_rows
        sp_off = sid * sp_rows

        def sp_read():
            return pltpu.make_async_copy(
                table_hbm.at[pl.ds(sp_tab, sp_rows)],
                spmem.at[pl.ds(sp_off, sp_rows)], srsem)

        def sp_write(b):
            return pltpu.make_async_copy(
                spmem.at[pl.ds(sp_off, sp_rows)],
                out_hbm.at[b, pl.ds(sp_tab, sp_rows)], swsem)

        @pl.when(sid < sp_workers)
        def _():
            sp_read().start()

        def start_read(c):
            return pltpu.async_copy(
                table_hbm.at[pl.ds(base + c * chunk, chunk)],
                bufs.at[c % nbuf], rsem)

        def tail_write(c, b):
            return pltpu.make_async_copy(
                bufs.at[c % nbuf],
                out_hbm.at[b, pl.ds(base + c * chunk, chunk)], lsem)

        reads = {0: start_read(0)}
        writes = {}
        tail_chunks = []
        for c in range(n_chunks):
            reads[c].wait()
            if c + 1 < n_chunks:
                old = c + 1 - nbuf
                if old >= 0:
                    for w in writes.pop(old):
                        w.wait()
                    if old in tail_chunks:
                        tail_chunks.remove(old)

                        @pl.when(sid >= tail_sid)
                        def _():
                            tail_write(old, 2).wait()
                            tail_write(old, 3).wait()
                reads[c + 1] = start_read(c + 1)
            writes[c] = [
                pltpu.async_copy(
                    bufs.at[c % nbuf],
                    out_hbm.at[b, pl.ds(base + c * chunk, chunk)], wsem)
                for b in range(2)
            ]
            tail_chunks.append(c)

            @pl.when(sid >= tail_sid)
            def _():
                tail_write(c, 2).start()
                tail_write(c, 3).start()

            if c == 1:
                @pl.when(sid < sp_workers)
                def _():
                    sp_read().wait()
                    sp_write(2).start()
                    sp_write(3).start()

        for c in sorted(writes):
            for w in writes[c]:
                w.wait()
        for c in tail_chunks:
            @pl.when(sid >= tail_sid)
            def _():
                tail_write(c, 2).wait()
                tail_write(c, 3).wait()

        @pl.when(sid < sp_workers)
        def _():
            sp_write(2).wait()
            sp_write(3).wait()

    return body(symbol_library)


# speedup vs baseline: 1.2249x; 1.2249x over previous
import functools

import jax
import jax.numpy as jnp
from jax import lax
from jax.experimental import pallas as pl
from jax.experimental.pallas import tpu as pltpu
from jax.experimental.pallas import tpu_sc as plsc


def kernel(x, symbol_library):
    batch, seq_len, d_model = x.shape
    num_workers = 32
    rows_per_worker = seq_len // num_workers
    chunks = (48, 48, 16, 16)
    assert sum(chunks) == rows_per_worker
    starts = [sum(chunks[:i]) for i in range(len(chunks))]
    n_chunks = len(chunks)
    nbuf = 2
    bufrows = max(chunks)

    mesh = plsc.VectorSubcoreMesh(core_axis_name="c", subcore_axis_name="s")

    @functools.partial(
        pl.kernel,
        mesh=mesh,
        out_type=jax.ShapeDtypeStruct((batch, seq_len, d_model), x.dtype),
        scratch_types=[
            pltpu.VMEM((nbuf, bufrows, d_model), jnp.float32),
            pltpu.SemaphoreType.DMA,
            pltpu.SemaphoreType.DMA,
        ],
    )
    def body(table_hbm, out_hbm, bufs, rsem, wsem):
        wid = lax.axis_index("s") * 2 + lax.axis_index("c")
        base = wid * rows_per_worker

        def start_read(c):
            return pltpu.async_copy(
                table_hbm.at[pl.ds(base + starts[c], chunks[c])],
                bufs.at[c % nbuf, pl.ds(0, chunks[c])], rsem)

        reads = {0: start_read(0)}
        writes = {}
        for c in range(n_chunks):
            reads[c].wait()
            if c + 1 < n_chunks:
                if c + 1 >= nbuf:
                    for w in writes.pop(c + 1 - nbuf):
                        w.wait()
                reads[c + 1] = start_read(c + 1)
            writes[c] = [
                pltpu.async_copy(
                    bufs.at[c % nbuf, pl.ds(0, chunks[c])],
                    out_hbm.at[b, pl.ds(base + starts[c], chunks[c])], wsem)
                for b in range(batch)
            ]
        for c in sorted(writes):
            for w in writes[c]:
                w.wait()

    return body(symbol_library)
